# baseline (speedup 1.0000x reference)
import functools

import jax
import jax.numpy as jnp
from jax import lax
from jax.experimental import pallas as pl
from jax.experimental.pallas import tpu as pltpu
from jax.experimental.pallas import tpu_sc as plsc

T_ROWS = 1000
T_PAD = 1024
D = 128
B = 16384

NC = 2
NS = 16
NW = NC * NS
B_PER_W = B // NW
IDX_CHUNK = 128
N_CHUNKS = B_PER_W // IDX_CHUNK
ROWS_PER_SUB = T_PAD // NS


def _mlp_body(table_ref, w1_ref, b1_ref, w2_ref, b2_ref, out_ref):
    h = jnp.dot(table_ref[...], w1_ref[...],
                preferred_element_type=jnp.float32) + b1_ref[...]
    h = h * jax.nn.sigmoid(h)
    out_ref[pl.ds(0, T_ROWS), :] = jnp.dot(
        h, w2_ref[...], preferred_element_type=jnp.float32) + b2_ref[...]


def _mlp_table(table, W1, b1, W2, b2):
    return pl.pallas_call(
        _mlp_body,
        out_shape=jax.ShapeDtypeStruct((T_PAD, D), jnp.float32),
    )(table, W1, b1.reshape(1, D), W2, b2.reshape(1, D))


def _gather_body(idx_hbm, tab_hbm, out_hbm, idx_v, rows_v, shared_tab, sem,
                 sem_w, sem_stage, sem_idx):
    sid = lax.axis_index("s")
    wid = sid * NC + lax.axis_index("c")
    base = wid * B_PER_W
    idx_cp = pltpu.async_copy(idx_hbm.at[wid], idx_v, sem_idx)
    idx_cp.wait()
    gathers = []
    for j in range(N_CHUNKS):
        gathers.append(pltpu.async_copy(
            tab_hbm.at[idx_v.at[j]],
            rows_v.at[pl.ds(j * IDX_CHUNK, IDX_CHUNK)],
            sem.at[j]))
    writes = []
    for j in range(N_CHUNKS):
        gathers[j].wait()
        writes.append(pltpu.async_copy(
            rows_v.at[pl.ds(j * IDX_CHUNK, IDX_CHUNK)],
            out_hbm.at[pl.ds(base + j * IDX_CHUNK, IDX_CHUNK)],
            sem_w))
    for w in writes:
        w.wait()


_gather = functools.partial(
    pl.kernel,
    mesh=plsc.VectorSubcoreMesh(
        core_axis_name="c", subcore_axis_name="s",
        num_cores=NC, num_subcores=NS),
    out_type=jax.ShapeDtypeStruct((B, D), jnp.float32),
    scratch_types=[
        pltpu.VMEM((N_CHUNKS, IDX_CHUNK), jnp.int32),
        pltpu.VMEM((B_PER_W, D), jnp.float32),
        pltpu.VMEM_SHARED((T_PAD, D), jnp.float32),
        pltpu.SemaphoreType.DMA((N_CHUNKS,)),
        pltpu.SemaphoreType.DMA,
        pltpu.SemaphoreType.DMA,
        pltpu.SemaphoreType.DMA,
    ],
)(_gather_body)


@jax.jit
def kernel(t, table, W1, b1, W2, b2):
    tab2 = _mlp_table(table, W1, b1, W2, b2)
    idx = t.astype(jnp.int32).reshape(NW, N_CHUNKS, IDX_CHUNK)
    return _gather(idx, tab2)


# speedup vs baseline: 1.2002x; 1.2002x over previous
import functools

import jax
import jax.numpy as jnp
from jax import lax
from jax.experimental import pallas as pl
from jax.experimental.pallas import tpu as pltpu
from jax.experimental.pallas import tpu_sc as plsc

T_ROWS = 1000
T_PAD = 1024
D = 128
B = 16384

NC = 2
NS = 16
NW = NC * NS
B_PER_W = B // NW
IDX_CHUNK = 128
N_CHUNKS = B_PER_W // IDX_CHUNK
ROWS_PER_SUB = T_PAD // NS


def _mlp_body(table_ref, w1_ref, b1_ref, w2_ref, b2_ref, out_ref):
    h = jnp.dot(table_ref[...], w1_ref[...],
                preferred_element_type=jnp.float32) + b1_ref[...]
    h = h * jax.nn.sigmoid(h)
    out_ref[pl.ds(0, T_ROWS), :] = jnp.dot(
        h, w2_ref[...], preferred_element_type=jnp.float32) + b2_ref[...]


def _mlp_table(table, W1, b1, W2, b2):
    return pl.pallas_call(
        _mlp_body,
        out_shape=jax.ShapeDtypeStruct((T_PAD, D), jnp.float32),
    )(table, W1, b1.reshape(1, D), W2, b2.reshape(1, D))


def _gather_body(idx_hbm, tab_hbm, out_hbm, idx_v, rows_v, shared_tab, sem,
                 sem_w, sem_stage, sem_idx):
    sid = lax.axis_index("s")
    wid = sid * NC + lax.axis_index("c")
    base = wid * B_PER_W
    stage = pltpu.async_copy(
        tab_hbm.at[pl.ds(sid * ROWS_PER_SUB, ROWS_PER_SUB)],
        shared_tab.at[pl.ds(sid * ROWS_PER_SUB, ROWS_PER_SUB)], sem_stage)
    idx_cp = pltpu.async_copy(idx_hbm.at[wid], idx_v, sem_idx)
    idx_cp.wait()
    gathers = [pltpu.async_copy(
        tab_hbm.at[idx_v.at[0]], rows_v.at[pl.ds(0, IDX_CHUNK)],
        sem.at[0])]
    stage.wait()
    plsc.subcore_barrier()
    for j in range(1, N_CHUNKS):
        gathers.append(pltpu.async_copy(
            shared_tab.at[idx_v.at[j]],
            rows_v.at[pl.ds(j * IDX_CHUNK, IDX_CHUNK)],
            sem.at[j]))
    writes = []
    for j in range(N_CHUNKS):
        gathers[j].wait()
        writes.append(pltpu.async_copy(
            rows_v.at[pl.ds(j * IDX_CHUNK, IDX_CHUNK)],
            out_hbm.at[pl.ds(base + j * IDX_CHUNK, IDX_CHUNK)],
            sem_w))
    for w in writes:
        w.wait()


_gather = functools.partial(
    pl.kernel,
    mesh=plsc.VectorSubcoreMesh(
        core_axis_name="c", subcore_axis_name="s",
        num_cores=NC, num_subcores=NS),
    out_type=jax.ShapeDtypeStruct((B, D), jnp.float32),
    scratch_types=[
        pltpu.VMEM((N_CHUNKS, IDX_CHUNK), jnp.int32),
        pltpu.VMEM((B_PER_W, D), jnp.float32),
        pltpu.VMEM_SHARED((T_PAD, D), jnp.float32),
        pltpu.SemaphoreType.DMA((N_CHUNKS,)),
        pltpu.SemaphoreType.DMA,
        pltpu.SemaphoreType.DMA,
        pltpu.SemaphoreType.DMA,
    ],
)(_gather_body)


@jax.jit
def kernel(t, table, W1, b1, W2, b2):
    tab2 = _mlp_table(table, W1, b1, W2, b2)
    idx = t.astype(jnp.int32).reshape(NW, N_CHUNKS, IDX_CHUNK)
    return _gather(idx, tab2)
